# baseline (speedup 1.0000x reference)
import functools

import jax
import jax.numpy as jnp
from jax.experimental import pallas as pl
from jax.experimental.pallas import tpu as pltpu
from jax.experimental.pallas import tpu_sc as plsc

BETA = 0.2
B, C, H, W = 8, 64, 64, 64
K, D = 1024, 64
N = B * H * W
BT = 512
NB = (H * W) // BT
GW = 256


def _vq_block(feat_ref, cbh_ref, y2_ref, idx_ref, loss_ref, acc_ref):
    b = pl.program_id(0)
    t = pl.program_id(1)
    x = feat_ref[...]
    cb_hi = cbh_ref[...]
    y2 = y2_ref[...]
    xs = (-2.0 * x).astype(jnp.bfloat16)
    scores = y2 + jax.lax.dot_general(
        cb_hi, xs, (((1,), (0,)), ((), ())),
        preferred_element_type=jnp.float32)
    smin = jnp.min(scores, axis=0)
    iota_k = jax.lax.broadcasted_iota(jnp.int32, (K, BT), 0)
    idx = jnp.min(jnp.where(scores == smin[None, :], iota_k, K), axis=0)
    idx_ref[0, 0, :] = idx

    part = jnp.sum(smin) + jnp.sum(x * x)

    @pl.when((b == 0) & (t == 0))
    def _init():
        acc_ref[0] = 0.0

    acc_ref[0] += part

    @pl.when((b == B - 1) & (t == NB - 1))
    def _fin():
        loss_ref[...] = jnp.full((1, 1), acc_ref[0] * ((1.0 + BETA) / (N * D)),
                                 dtype=jnp.float32)


def _tc_stage(feat2d, cb_hi, y2):
    return pl.pallas_call(
        _vq_block,
        grid=(B, NB),
        in_specs=[
            pl.BlockSpec((C, BT), lambda b, t: (b, t)),
            pl.BlockSpec((K, D), lambda b, t: (0, 0)),
            pl.BlockSpec((K, 1), lambda b, t: (0, 0)),
        ],
        out_specs=[
            pl.BlockSpec((1, 1, BT), lambda b, t: (b * NB + t, 0, 0)),
            pl.BlockSpec((1, 1), lambda b, t: (0, 0)),
        ],
        out_shape=[
            jax.ShapeDtypeStruct((B * NB, 1, BT), jnp.int32),
            jax.ShapeDtypeStruct((1, 1), jnp.float32),
        ],
        scratch_shapes=[pltpu.SMEM((1,), jnp.float32)],
    )(feat2d, cb_hi, y2)


_NW = 32
_BPW = N // _NW


_CH = 512


def _sc_gather(cb128, idx_flat):
    mesh = plsc.VectorSubcoreMesh(core_axis_name="c", subcore_axis_name="s")

    @functools.partial(
        pl.kernel, mesh=mesh,
        out_type=jax.ShapeDtypeStruct((N, 128), jnp.float32),
        scratch_types=[
            pltpu.VMEM((_BPW,), jnp.int32),
            pltpu.VMEM((_CH, 128), jnp.float32),
            pltpu.SemaphoreType.DMA,
        ],
    )
    def gather_kernel(cb_hbm, idx_hbm, out_hbm, idx_v, rows_v, sem):
        wid = jax.lax.axis_index("s") * 2 + jax.lax.axis_index("c")
        base = wid * _BPW
        pltpu.sync_copy(idx_hbm.at[pl.ds(base, _BPW)], idx_v)

        @pl.loop(0, _BPW, step=_CH)
        def _(c):
            pltpu.async_copy(cb_hbm.at[idx_v.at[pl.ds(c, _CH)]], rows_v,
                             sem).wait()
            pltpu.sync_copy(rows_v, out_hbm.at[pl.ds(base + c, _CH)])

    return gather_kernel(cb128, idx_flat)


@jax.jit
def kernel(features, codebook):
    feat2d = features.reshape(B * C, H * W)
    y2 = jnp.sum(codebook * codebook, axis=1, keepdims=True)
    cb_hi = codebook.astype(jnp.bfloat16)
    idx, loss = _tc_stage(feat2d, cb_hi, y2)
    cb128 = jnp.concatenate(
        [codebook, jnp.zeros((K, 128 - D), jnp.float32)], axis=1)
    out128 = _sc_gather(cb128, idx.reshape(N))
    return out128[:, :D].reshape(B, C, H, W), loss[0, 0]
